# baseline (speedup 1.0000x reference)
import functools

import jax
import jax.numpy as jnp
from jax import lax
from jax.experimental import pallas as pl
from jax.experimental.pallas import tpu as pltpu
from jax.experimental.pallas import tpu_sc as plsc

N = 10000
E = 160000
D = 256
H = 32

NW = 32
CH = 128
NCHUNK = 40
EPT = CH * NCHUNK
EP = EPT * NW
NP = 10240
RPT = NP // 16
DUMMY = NP


def _deg_body(ei3, zeros1, out, dst_v, ones_v, acc):
    c = lax.axis_index("c")
    s = lax.axis_index("s")
    wid = c * 16 + s
    pltpu.sync_copy(ei3.at[1, wid], dst_v)
    for i in range(CH // 16):
        ones_v[pl.ds(16 * i, 16)] = jnp.full((16,), 1.0, jnp.float32)
    pltpu.sync_copy(zeros1.at[pl.ds(s * RPT, RPT)], acc.at[pl.ds(s * RPT, RPT)])
    plsc.subcore_barrier()

    def body(j, _):
        pltpu.sync_copy(ones_v, acc.at[dst_v.at[j]], add=True)
        return ()

    lax.fori_loop(0, NCHUNK, body, ())
    plsc.subcore_barrier()
    pltpu.sync_copy(acc.at[pl.ds(s * RPT, RPT)], out.at[c, pl.ds(s * RPT, RPT)])


def _seg_body(table, ei3, zeros2, out, src_v, dst_v,
              rows0, rows1, rows2, rows3,
              acc, tab_sh, g0, g1, g2, g3, s0, s1, s2, s3):
    c = lax.axis_index("c")
    s = lax.axis_index("s")
    wid = c * 16 + s
    st0 = pltpu.async_copy(ei3.at[0, wid], src_v, g0)
    st1 = pltpu.async_copy(ei3.at[1, wid], dst_v, g1)
    st2 = pltpu.async_copy(zeros2.at[pl.ds(s * RPT, RPT)],
                           acc.at[pl.ds(s * RPT, RPT)], g2)
    st3 = pltpu.async_copy(table.at[pl.ds(s * RPT, RPT), pl.ds(0, H)],
                           tab_sh.at[pl.ds(s * RPT, RPT)], g3)
    st0.wait()
    st1.wait()
    st2.wait()
    st3.wait()
    plsc.subcore_barrier()

    bufs = (rows0, rows1, rows2, rows3)
    gsem = (g0, g1, g2, g3)
    ssem = (s0, s1, s2, s3)
    NB = 4
    NO = NCHUNK // NB

    pltpu.async_copy(tab_sh.at[src_v.at[0]], bufs[0], gsem[0])
    pltpu.async_copy(tab_sh.at[src_v.at[1]], bufs[1], gsem[1])

    def body(o, _):
        for b in range(NB):
            j = NB * o + b
            bb = (b + 2) % NB
            pltpu.make_async_copy(tab_sh.at[src_v.at[j]], bufs[b],
                                  gsem[b]).wait()
            pltpu.async_copy(bufs[b], acc.at[dst_v.at[j]], ssem[b], add=True)

            def wait_sc():
                pltpu.make_async_copy(bufs[bb], acc.at[dst_v.at[j]],
                                      ssem[bb]).wait()

            def start_g():
                pltpu.async_copy(tab_sh.at[src_v.at[j + 2]], bufs[bb],
                                 gsem[bb])

            if b < 2:
                pl.when(o > 0)(wait_sc)
                start_g()
            else:
                wait_sc()
                pl.when(o < NO - 1)(start_g)
        return ()

    lax.fori_loop(0, NO, body, ())
    pltpu.make_async_copy(bufs[2], acc.at[dst_v.at[NCHUNK - 2]],
                          ssem[2]).wait()
    pltpu.make_async_copy(bufs[3], acc.at[dst_v.at[NCHUNK - 1]],
                          ssem[3]).wait()
    plsc.subcore_barrier()
    pltpu.sync_copy(acc.at[pl.ds(s * RPT, RPT)],
                    out.at[c, pl.ds(s * RPT, RPT), pl.ds(0, H)])


@functools.lru_cache(maxsize=None)
def _sc_kernels():
    mesh = plsc.VectorSubcoreMesh(core_axis_name="c", subcore_axis_name="s")
    deg = pl.kernel(
        _deg_body,
        out_type=jax.ShapeDtypeStruct((2, NP), jnp.float32),
        mesh=mesh,
        scratch_types=[
            pltpu.VMEM((NCHUNK, CH), jnp.int32),
            pltpu.VMEM((CH,), jnp.float32),
            pltpu.VMEM_SHARED((NP + 8,), jnp.float32),
        ],
    )
    seg = pl.kernel(
        _seg_body,
        out_type=jax.ShapeDtypeStruct((2, NP, 128), jnp.float32),
        mesh=mesh,
        compiler_params=pltpu.CompilerParams(use_tc_tiling_on_sc=False),
        scratch_types=[
            pltpu.VMEM((NCHUNK, CH), jnp.int32),
            pltpu.VMEM((NCHUNK, CH), jnp.int32),
            pltpu.VMEM((CH, H), jnp.float32),
            pltpu.VMEM((CH, H), jnp.float32),
            pltpu.VMEM((CH, H), jnp.float32),
            pltpu.VMEM((CH, H), jnp.float32),
            pltpu.VMEM_SHARED((NP + 8, H), jnp.float32),
            pltpu.VMEM_SHARED((NP + 8, H), jnp.float32),
        ] + [pltpu.SemaphoreType.DMA] * 8,
    )
    return deg, seg


BN = 1280
GR = NP // BN


def _mask32(v):
    lanes = lax.broadcasted_iota(jnp.int32, v.shape, v.ndim - 1)
    return jnp.where(lanes < H, v, 0.0)


def _tc1_body(x_ref, w1p_ref, degp_ref, hs_ref, disb_ref):
    deg = degp_ref[0:1, :] + degp_ref[1:2, :] + 1.0
    dis_col = jnp.transpose(lax.rsqrt(deg))
    disb_ref[...] = jnp.broadcast_to(dis_col, (BN, H))
    dis128 = jnp.broadcast_to(dis_col, (BN, 128))
    xw = jnp.dot(x_ref[...], w1p_ref[...], preferred_element_type=jnp.float32)
    hs_ref[...] = xw * dis128


def _tc2_body(p1_ref, hs1_ref, disb_ref, b1p_ref, w2p_ref, out_ref):
    dis128 = jnp.broadcast_to(disb_ref[:, 0:1], (BN, 128))
    p1 = _mask32(p1_ref[0] + p1_ref[1])
    h1 = jnp.maximum(dis128 * (p1 + hs1_ref[...]) + b1p_ref[...], 0.0)
    out_ref[...] = jnp.dot(h1, w2p_ref[...],
                           preferred_element_type=jnp.float32) * dis128


def _tc3_body(p2_ref, hs2_ref, disb_ref, b2p_ref, wap_ref, bap_ref,
              wfp_ref, bfp_ref, wdp_ref, bd_ref, out_ref, acc_ref):
    i = pl.program_id(0)
    dis128 = jnp.broadcast_to(disb_ref[:, 0:1], (BN, 128))
    p2 = _mask32(p2_ref[0] + p2_ref[1])
    h2 = jnp.maximum(dis128 * (p2 + hs2_ref[...]) + b2p_ref[...], 0.0)
    za = jnp.dot(h2, wap_ref[...], preferred_element_type=jnp.float32) + bap_ref[...]
    attn = 1.0 / (1.0 + jnp.exp(-za))
    feat = jnp.dot(h2, wfp_ref[...], preferred_element_type=jnp.float32) + bfp_ref[...]
    rows = i * BN + lax.broadcasted_iota(jnp.int32, (BN, 128), 0)
    af = jnp.where(rows < N, attn * feat, 0.0)
    psum = jnp.sum(af, axis=0, keepdims=True)

    @pl.when(i == 0)
    def _():
        acc_ref[...] = psum

    @pl.when(i > 0)
    def _():
        acc_ref[...] = acc_ref[...] + psum

    @pl.when(i == GR - 1)
    def _():
        out_ref[...] = jnp.dot(acc_ref[...], wdp_ref[...],
                               preferred_element_type=jnp.float32) + bd_ref[...]


def kernel(x, edge_index, W1, b1, W2, b2, Wa, ba, Wf, bf, Wd, bd):
    f32 = jnp.float32
    ei3 = jnp.pad(edge_index, ((0, 0), (0, EP - E)),
                  constant_values=DUMMY).reshape(2, NW, NCHUNK, CH)
    zeros1 = jnp.zeros((NP,), f32)
    zeros2 = jnp.zeros((NP, H), f32)

    W1p = jnp.pad(W1, ((0, 0), (0, 128 - H)))
    W2p = jnp.pad(W2, ((0, 128 - H), (0, 128 - H)))
    Wap = jnp.pad(Wa, ((0, 128 - H), (0, 128 - H)))
    Wfp = jnp.pad(Wf, ((0, 128 - H), (0, 128 - H)))
    Wdp = jnp.pad(Wd, ((0, 128 - H), (0, 0)))
    b1p = jnp.pad(b1, (0, 128 - H)).reshape(1, 128)
    b2p = jnp.pad(b2, (0, 128 - H)).reshape(1, 128)
    bap = jnp.pad(ba, (0, 128 - H)).reshape(1, 128)
    bfp = jnp.pad(bf, (0, 128 - H)).reshape(1, 128)

    _deg_kernel, _seg_kernel = _sc_kernels()
    degp = _deg_kernel(ei3, zeros1)

    row_blk = pl.BlockSpec((BN, 128), lambda i: (i, 0))
    disb_blk = pl.BlockSpec((BN, H), lambda i: (i, 0))
    p_blk = pl.BlockSpec((2, BN, 128), lambda i: (0, i, 0))
    fix = lambda shape: pl.BlockSpec(shape, lambda i: tuple(0 for _ in shape))

    hs1, disb = pl.pallas_call(
        _tc1_body,
        grid=(GR,),
        in_specs=[pl.BlockSpec((BN, D), lambda i: (i, 0)), fix((D, 128)),
                  pl.BlockSpec((2, BN), lambda i: (0, i))],
        out_specs=[row_blk, disb_blk],
        out_shape=[jax.ShapeDtypeStruct((NP, 128), f32),
                   jax.ShapeDtypeStruct((NP, H), f32)],
    )(x, W1p, degp)

    P1 = _seg_kernel(hs1, ei3, zeros2)

    hs2 = pl.pallas_call(
        _tc2_body,
        grid=(GR,),
        in_specs=[p_blk, row_blk, disb_blk, fix((1, 128)), fix((128, 128))],
        out_specs=row_blk,
        out_shape=jax.ShapeDtypeStruct((NP, 128), f32),
    )(P1, hs1, disb, b1p, W2p)

    P2 = _seg_kernel(hs2, ei3, zeros2)

    out = pl.pallas_call(
        _tc3_body,
        grid=(GR,),
        in_specs=[p_blk, row_blk, disb_blk, fix((1, 128)), fix((128, 128)),
                  fix((1, 128)), fix((128, 128)), fix((1, 128)),
                  fix((128, 1)), fix((1, 1))],
        out_specs=fix((1, 1)),
        out_shape=jax.ShapeDtypeStruct((1, 1), f32),
        scratch_shapes=[pltpu.VMEM((1, 128), f32)],
    )(P2, hs2, disb, b2p, Wap, bap, Wfp, bfp, Wdp, bd.reshape(1, 1))
    return out


# speedup vs baseline: 1.0205x; 1.0205x over previous
import functools

import jax
import jax.numpy as jnp
from jax import lax
from jax.experimental import pallas as pl
from jax.experimental.pallas import tpu as pltpu
from jax.experimental.pallas import tpu_sc as plsc

N = 10000
E = 160000
D = 256
H = 32

NW = 32
CH = 128
NCHUNK = 40
EPT = CH * NCHUNK
EP = EPT * NW
NP = 10240
RPT = NP // 16
DUMMY = NP


def _deg_body(ei3, zeros1, out, dst_v, ones_v, acc, dsem):
    c = lax.axis_index("c")
    s = lax.axis_index("s")
    wid = c * 16 + s
    pltpu.sync_copy(ei3.at[1, wid], dst_v)
    for i in range(CH // 16):
        ones_v[pl.ds(16 * i, 16)] = jnp.full((16,), 1.0, jnp.float32)
    pltpu.sync_copy(zeros1.at[pl.ds(s * RPT, RPT)], acc.at[pl.ds(s * RPT, RPT)])
    plsc.subcore_barrier()

    def body(o, _):
        for b in range(8):
            pltpu.async_copy(ones_v, acc.at[dst_v.at[8 * o + b]], dsem,
                             add=True)
        for b in range(8):
            pltpu.make_async_copy(ones_v, acc.at[dst_v.at[8 * o + b]],
                                  dsem).wait()
        return ()

    lax.fori_loop(0, NCHUNK // 8, body, ())
    plsc.subcore_barrier()
    pltpu.sync_copy(acc.at[pl.ds(s * RPT, RPT)], out.at[c, pl.ds(s * RPT, RPT)])


def _seg_body(table, ei3, zeros2, out, src_v, dst_v,
              rows0, rows1, rows2, rows3, rows4, rows5, rows6, rows7,
              acc, tab_sh, g0, g1, g2, g3, g4, g5, g6, g7,
              s0, s1, s2, s3, s4, s5, s6, s7):
    c = lax.axis_index("c")
    s = lax.axis_index("s")
    wid = c * 16 + s
    st0 = pltpu.async_copy(ei3.at[0, wid], src_v, g0)
    st1 = pltpu.async_copy(ei3.at[1, wid], dst_v, g1)
    st2 = pltpu.async_copy(zeros2.at[pl.ds(s * RPT, RPT)],
                           acc.at[pl.ds(s * RPT, RPT)], g2)
    st3 = pltpu.async_copy(table.at[pl.ds(s * RPT, RPT), pl.ds(0, H)],
                           tab_sh.at[pl.ds(s * RPT, RPT)], g3)
    st0.wait()
    st1.wait()
    st2.wait()
    st3.wait()
    plsc.subcore_barrier()

    bufs = (rows0, rows1, rows2, rows3, rows4, rows5, rows6, rows7)
    gsem = (g0, g1, g2, g3, g4, g5, g6, g7)
    ssem = (s0, s1, s2, s3, s4, s5, s6, s7)
    NB = 8
    HB = NB // 2
    NO = NCHUNK // NB

    for k in range(HB):
        pltpu.async_copy(tab_sh.at[src_v.at[k]], bufs[k], gsem[k])

    def body(o, _):
        for b in range(NB):
            j = NB * o + b
            bb = (b + HB) % NB
            pltpu.make_async_copy(tab_sh.at[src_v.at[j]], bufs[b],
                                  gsem[b]).wait()
            pltpu.async_copy(bufs[b], acc.at[dst_v.at[j]], ssem[b], add=True)

            def wait_sc():
                pltpu.make_async_copy(bufs[bb], acc.at[dst_v.at[j]],
                                      ssem[bb]).wait()

            def start_g():
                pltpu.async_copy(tab_sh.at[src_v.at[j + HB]], bufs[bb],
                                 gsem[bb])

            if b < HB:
                pl.when(o > 0)(wait_sc)
                start_g()
            else:
                wait_sc()
                pl.when(o < NO - 1)(start_g)
        return ()

    lax.fori_loop(0, NO, body, ())
    for k in range(HB):
        j = NCHUNK - HB + k
        pltpu.make_async_copy(bufs[j % NB], acc.at[dst_v.at[j]],
                              ssem[j % NB]).wait()
    plsc.subcore_barrier()
    pltpu.sync_copy(acc.at[pl.ds(s * RPT, RPT)],
                    out.at[c, pl.ds(s * RPT, RPT), pl.ds(0, H)])


@functools.lru_cache(maxsize=None)
def _sc_kernels():
    mesh = plsc.VectorSubcoreMesh(core_axis_name="c", subcore_axis_name="s")
    deg = pl.kernel(
        _deg_body,
        out_type=jax.ShapeDtypeStruct((2, NP), jnp.float32),
        mesh=mesh,
        scratch_types=[
            pltpu.VMEM((NCHUNK, CH), jnp.int32),
            pltpu.VMEM((CH,), jnp.float32),
            pltpu.VMEM_SHARED((NP + 8,), jnp.float32),
            pltpu.SemaphoreType.DMA,
        ],
    )
    seg = pl.kernel(
        _seg_body,
        out_type=jax.ShapeDtypeStruct((2, NP, 128), jnp.float32),
        mesh=mesh,
        compiler_params=pltpu.CompilerParams(use_tc_tiling_on_sc=False),
        scratch_types=[
            pltpu.VMEM((NCHUNK, CH), jnp.int32),
            pltpu.VMEM((NCHUNK, CH), jnp.int32),
        ] + [pltpu.VMEM((CH, H), jnp.float32)] * 8 + [
            pltpu.VMEM_SHARED((NP + 8, H), jnp.float32),
            pltpu.VMEM_SHARED((NP + 8, H), jnp.float32),
        ] + [pltpu.SemaphoreType.DMA] * 16,
    )
    return deg, seg


def _mask32(v):
    lanes = lax.broadcasted_iota(jnp.int32, v.shape, v.ndim - 1)
    return jnp.where(lanes < H, v, 0.0)


def _tc1_body(x_ref, w1p_ref, degp_ref, hs_ref, disb_ref):
    deg = degp_ref[0:1, :] + degp_ref[1:2, :] + 1.0
    dis_col = jnp.transpose(lax.rsqrt(deg))
    disb_ref[...] = jnp.broadcast_to(dis_col, (NP, H))
    dis128 = jnp.broadcast_to(dis_col[:N], (N, 128))
    xw = jnp.dot(x_ref[...], w1p_ref[...], preferred_element_type=jnp.float32)
    hs_ref[:N] = xw * dis128
    hs_ref[N:] = jnp.zeros((NP - N, 128), jnp.float32)


def _tc2_body(p1_ref, hs1_ref, disb_ref, b1p_ref, w2p_ref, out_ref):
    dis128 = jnp.broadcast_to(disb_ref[:N, 0:1], (N, 128))
    p1 = _mask32(p1_ref[0, :N] + p1_ref[1, :N])
    h1 = jnp.maximum(dis128 * (p1 + hs1_ref[:N]) + b1p_ref[...], 0.0)
    out_ref[:N] = jnp.dot(h1, w2p_ref[...],
                          preferred_element_type=jnp.float32) * dis128
    out_ref[N:] = jnp.zeros((NP - N, 128), jnp.float32)


def _tc3_body(p2_ref, hs2_ref, disb_ref, b2p_ref, wap_ref, bap_ref,
              wfp_ref, bfp_ref, wdp_ref, bd_ref, out_ref):
    dis128 = jnp.broadcast_to(disb_ref[:N, 0:1], (N, 128))
    p2 = _mask32(p2_ref[0, :N] + p2_ref[1, :N])
    h2 = jnp.maximum(dis128 * (p2 + hs2_ref[:N]) + b2p_ref[...], 0.0)
    za = jnp.dot(h2, wap_ref[...], preferred_element_type=jnp.float32) + bap_ref[...]
    attn = 1.0 / (1.0 + jnp.exp(-za))
    feat = jnp.dot(h2, wfp_ref[...], preferred_element_type=jnp.float32) + bfp_ref[...]
    pooled = jnp.sum(attn * feat, axis=0, keepdims=True)
    out_ref[...] = jnp.dot(pooled, wdp_ref[...],
                           preferred_element_type=jnp.float32) + bd_ref[...]


def kernel(x, edge_index, W1, b1, W2, b2, Wa, ba, Wf, bf, Wd, bd):
    f32 = jnp.float32
    ei3 = jnp.pad(edge_index, ((0, 0), (0, EP - E)),
                  constant_values=DUMMY).reshape(2, NW, NCHUNK, CH)
    zeros1 = jnp.zeros((NP,), f32)
    zeros2 = jnp.zeros((NP, H), f32)

    W1p = jnp.pad(W1, ((0, 0), (0, 128 - H)))
    W2p = jnp.pad(W2, ((0, 128 - H), (0, 128 - H)))
    Wap = jnp.pad(Wa, ((0, 128 - H), (0, 128 - H)))
    Wfp = jnp.pad(Wf, ((0, 128 - H), (0, 128 - H)))
    Wdp = jnp.pad(Wd, ((0, 128 - H), (0, 0)))
    b1p = jnp.pad(b1, (0, 128 - H)).reshape(1, 128)
    b2p = jnp.pad(b2, (0, 128 - H)).reshape(1, 128)
    bap = jnp.pad(ba, (0, 128 - H)).reshape(1, 128)
    bfp = jnp.pad(bf, (0, 128 - H)).reshape(1, 128)

    _deg_kernel, _seg_kernel = _sc_kernels()
    degp = _deg_kernel(ei3, zeros1)

    hs1, disb = pl.pallas_call(
        _tc1_body,
        out_shape=[jax.ShapeDtypeStruct((NP, 128), f32),
                   jax.ShapeDtypeStruct((NP, H), f32)],
    )(x, W1p, degp)

    P1 = _seg_kernel(hs1, ei3, zeros2)

    hs2 = pl.pallas_call(
        _tc2_body,
        out_shape=jax.ShapeDtypeStruct((NP, 128), f32),
    )(P1, hs1, disb, b1p, W2p)

    P2 = _seg_kernel(hs2, ei3, zeros2)

    out = pl.pallas_call(
        _tc3_body,
        out_shape=jax.ShapeDtypeStruct((1, 1), f32),
    )(P2, hs2, disb, b2p, Wap, bap, Wfp, bfp, Wdp, bd.reshape(1, 1))
    return out


# speedup vs baseline: 1.0243x; 1.0038x over previous
import functools

import jax
import jax.numpy as jnp
from jax import lax
from jax.experimental import pallas as pl
from jax.experimental.pallas import tpu as pltpu
from jax.experimental.pallas import tpu_sc as plsc

N = 10000
E = 160000
D = 256
H = 32

NW = 32
CH = 128
NCHUNK = 40
EPT = CH * NCHUNK
EP = EPT * NW
NP = 10240
RPT = NP // 16
DUMMY = NP


def _deg_body(ei3, zeros1, out, dst_v, ones_v, acc):
    c = lax.axis_index("c")
    s = lax.axis_index("s")
    wid = c * 16 + s
    pltpu.sync_copy(ei3.at[1, wid], dst_v)
    for i in range(CH // 16):
        ones_v[pl.ds(16 * i, 16)] = jnp.full((16,), 1.0, jnp.float32)
    pltpu.sync_copy(zeros1.at[pl.ds(s * RPT, RPT)], acc.at[pl.ds(s * RPT, RPT)])
    plsc.subcore_barrier()

    def body(j, _):
        pltpu.sync_copy(ones_v, acc.at[dst_v.at[j]], add=True)
        return ()

    lax.fori_loop(0, NCHUNK, body, ())
    plsc.subcore_barrier()
    pltpu.sync_copy(acc.at[pl.ds(s * RPT, RPT)], out.at[c, pl.ds(s * RPT, RPT)])


def _seg_body(table, ei3, zeros2, out, src_v, dst_v,
              rows0, rows1, rows2, rows3,
              acc, tab_sh, g0, g1, g2, g3, s0, s1, s2, s3):
    c = lax.axis_index("c")
    s = lax.axis_index("s")
    wid = c * 16 + s
    st0 = pltpu.async_copy(ei3.at[0, wid], src_v, g0)
    st1 = pltpu.async_copy(ei3.at[1, wid], dst_v, g1)
    st2 = pltpu.async_copy(zeros2.at[pl.ds(s * RPT, RPT)],
                           acc.at[pl.ds(s * RPT, RPT)], g2)
    st3 = pltpu.async_copy(table.at[pl.ds(s * RPT, RPT), pl.ds(0, H)],
                           tab_sh.at[pl.ds(s * RPT, RPT)], g3)
    st0.wait()
    st1.wait()
    st2.wait()
    st3.wait()
    plsc.subcore_barrier()

    bufs = (rows0, rows1, rows2, rows3)
    gsem = (g0, g1, g2, g3)
    ssem = (s0, s1, s2, s3)
    NB = 4
    NO = NCHUNK // NB

    pltpu.async_copy(tab_sh.at[src_v.at[0]], bufs[0], gsem[0])
    pltpu.async_copy(tab_sh.at[src_v.at[1]], bufs[1], gsem[1])

    def body(o, _):
        for b in range(NB):
            j = NB * o + b
            bb = (b + 2) % NB
            pltpu.make_async_copy(tab_sh.at[src_v.at[j]], bufs[b],
                                  gsem[b]).wait()
            pltpu.async_copy(bufs[b], acc.at[dst_v.at[j]], ssem[b], add=True)

            def wait_sc():
                pltpu.make_async_copy(bufs[bb], acc.at[dst_v.at[j]],
                                      ssem[bb]).wait()

            def start_g():
                pltpu.async_copy(tab_sh.at[src_v.at[j + 2]], bufs[bb],
                                 gsem[bb])

            if b < 2:
                pl.when(o > 0)(wait_sc)
                start_g()
            else:
                wait_sc()
                pl.when(o < NO - 1)(start_g)
        return ()

    lax.fori_loop(0, NO, body, ())
    pltpu.make_async_copy(bufs[2], acc.at[dst_v.at[NCHUNK - 2]],
                          ssem[2]).wait()
    pltpu.make_async_copy(bufs[3], acc.at[dst_v.at[NCHUNK - 1]],
                          ssem[3]).wait()
    plsc.subcore_barrier()
    pltpu.sync_copy(acc.at[pl.ds(s * RPT, RPT)],
                    out.at[c, pl.ds(s * RPT, RPT), pl.ds(0, H)])


@functools.lru_cache(maxsize=None)
def _sc_kernels():
    mesh = plsc.VectorSubcoreMesh(core_axis_name="c", subcore_axis_name="s")
    deg = pl.kernel(
        _deg_body,
        out_type=jax.ShapeDtypeStruct((2, NP), jnp.float32),
        mesh=mesh,
        scratch_types=[
            pltpu.VMEM((NCHUNK, CH), jnp.int32),
            pltpu.VMEM((CH,), jnp.float32),
            pltpu.VMEM_SHARED((NP + 8,), jnp.float32),
        ],
    )
    seg = pl.kernel(
        _seg_body,
        out_type=jax.ShapeDtypeStruct((2, NP, 128), jnp.float32),
        mesh=mesh,
        compiler_params=pltpu.CompilerParams(use_tc_tiling_on_sc=False),
        scratch_types=[
            pltpu.VMEM((NCHUNK, CH), jnp.int32),
            pltpu.VMEM((NCHUNK, CH), jnp.int32),
            pltpu.VMEM((CH, H), jnp.float32),
            pltpu.VMEM((CH, H), jnp.float32),
            pltpu.VMEM((CH, H), jnp.float32),
            pltpu.VMEM((CH, H), jnp.float32),
            pltpu.VMEM_SHARED((NP + 8, H), jnp.float32),
            pltpu.VMEM_SHARED((NP + 8, H), jnp.float32),
        ] + [pltpu.SemaphoreType.DMA] * 8,
    )
    return deg, seg


def _mask32(v):
    lanes = lax.broadcasted_iota(jnp.int32, v.shape, v.ndim - 1)
    return jnp.where(lanes < H, v, 0.0)


def _tc1_body(x_ref, w1p_ref, degp_ref, hs_ref, disb_ref):
    deg = degp_ref[0:1, :] + degp_ref[1:2, :] + 1.0
    dis_col = jnp.transpose(lax.rsqrt(deg))
    disb_ref[...] = jnp.broadcast_to(dis_col, (NP, H))
    dis128 = jnp.broadcast_to(dis_col[:N], (N, 128))
    xw = jnp.dot(x_ref[...], w1p_ref[...], preferred_element_type=jnp.float32)
    hs_ref[:N] = xw * dis128
    hs_ref[N:] = jnp.zeros((NP - N, 128), jnp.float32)


def _tc2_body(p1_ref, hs1_ref, disb_ref, b1p_ref, w2p_ref, out_ref):
    dis128 = jnp.broadcast_to(disb_ref[:N, 0:1], (N, 128))
    p1 = _mask32(p1_ref[0, :N] + p1_ref[1, :N])
    h1 = jnp.maximum(dis128 * (p1 + hs1_ref[:N]) + b1p_ref[...], 0.0)
    out_ref[:N] = jnp.dot(h1, w2p_ref[...],
                          preferred_element_type=jnp.float32) * dis128
    out_ref[N:] = jnp.zeros((NP - N, 128), jnp.float32)


def _tc3_body(p2_ref, hs2_ref, disb_ref, b2p_ref, wap_ref, bap_ref,
              wfp_ref, bfp_ref, wdp_ref, bd_ref, out_ref):
    dis128 = jnp.broadcast_to(disb_ref[:N, 0:1], (N, 128))
    p2 = _mask32(p2_ref[0, :N] + p2_ref[1, :N])
    h2 = jnp.maximum(dis128 * (p2 + hs2_ref[:N]) + b2p_ref[...], 0.0)
    za = jnp.dot(h2, wap_ref[...], preferred_element_type=jnp.float32) + bap_ref[...]
    attn = 1.0 / (1.0 + jnp.exp(-za))
    feat = jnp.dot(h2, wfp_ref[...], preferred_element_type=jnp.float32) + bfp_ref[...]
    pooled = jnp.sum(attn * feat, axis=0, keepdims=True)
    out_ref[...] = jnp.dot(pooled, wdp_ref[...],
                           preferred_element_type=jnp.float32) + bd_ref[...]


def kernel(x, edge_index, W1, b1, W2, b2, Wa, ba, Wf, bf, Wd, bd):
    f32 = jnp.float32
    ei3 = jnp.pad(edge_index, ((0, 0), (0, EP - E)),
                  constant_values=DUMMY).reshape(2, NW, NCHUNK, CH)
    zeros1 = jnp.zeros((NP,), f32)
    zeros2 = jnp.zeros((NP, H), f32)

    W1p = jnp.pad(W1, ((0, 0), (0, 128 - H)))
    W2p = jnp.pad(W2, ((0, 128 - H), (0, 128 - H)))
    Wap = jnp.pad(Wa, ((0, 128 - H), (0, 128 - H)))
    Wfp = jnp.pad(Wf, ((0, 128 - H), (0, 128 - H)))
    Wdp = jnp.pad(Wd, ((0, 128 - H), (0, 0)))
    b1p = jnp.pad(b1, (0, 128 - H)).reshape(1, 128)
    b2p = jnp.pad(b2, (0, 128 - H)).reshape(1, 128)
    bap = jnp.pad(ba, (0, 128 - H)).reshape(1, 128)
    bfp = jnp.pad(bf, (0, 128 - H)).reshape(1, 128)

    _deg_kernel, _seg_kernel = _sc_kernels()
    degp = _deg_kernel(ei3, zeros1)

    hs1, disb = pl.pallas_call(
        _tc1_body,
        out_shape=[jax.ShapeDtypeStruct((NP, 128), f32),
                   jax.ShapeDtypeStruct((NP, H), f32)],
    )(x, W1p, degp)

    P1 = _seg_kernel(hs1, ei3, zeros2)

    hs2 = pl.pallas_call(
        _tc2_body,
        out_shape=jax.ShapeDtypeStruct((NP, 128), f32),
    )(P1, hs1, disb, b1p, W2p)

    P2 = _seg_kernel(hs2, ei3, zeros2)

    out = pl.pallas_call(
        _tc3_body,
        out_shape=jax.ShapeDtypeStruct((1, 1), f32),
    )(P2, hs2, disb, b2p, Wap, bap, Wfp, bfp, Wdp, bd.reshape(1, 1))
    return out
